# baseline (speedup 1.0000x reference)
import functools

import jax
import jax.numpy as jnp
from jax import lax
from jax.experimental import pallas as pl
from jax.experimental.pallas import tpu as pltpu
from jax.experimental.pallas import tpu_sc as plsc

NUM_NODES = 100000
MEM_DIM = 128
TIME_DIM = 100
B = 100000
HIDDEN = 100
OUT = 3

NC = 2
NS = 16
NW = NC * NS
C = 112
NCHUNK = 28
CH = C * NCHUNK
B_PAD = CH * NW

TB = 2048


def _sc_gather(src2, dst2, mem_hbm, lu_hbm):
    mesh = plsc.VectorSubcoreMesh(core_axis_name="c", subcore_axis_name="s")

    @functools.partial(
        pl.kernel,
        mesh=mesh,
        out_type=[
            jax.ShapeDtypeStruct((B_PAD, MEM_DIM), jnp.float32),
            jax.ShapeDtypeStruct((B_PAD, MEM_DIM), jnp.float32),
            jax.ShapeDtypeStruct((B_PAD,), jnp.float32),
        ],
        scratch_types=[
            pltpu.VMEM((NCHUNK, C), jnp.int32),
            pltpu.VMEM((NCHUNK, C), jnp.int32),
            pltpu.VMEM((2, C, MEM_DIM), jnp.float32),
            pltpu.VMEM((2, C, MEM_DIM), jnp.float32),
            pltpu.VMEM((2, C), jnp.float32),
            pltpu.SemaphoreType.DMA,
            pltpu.SemaphoreType.DMA,
        ],
    )
    def k(src_hbm, dst_hbm, table_hbm, lu_src_hbm, srcm_out, dstm_out, lu_out,
          sidx, didx, buf_s, buf_d, buf_lu, sem0, sem1):
        cid = lax.axis_index("c")
        sid = lax.axis_index("s")
        wid = sid * NC + cid
        base = wid * CH
        sems = (sem0, sem1)

        pltpu.sync_copy(src_hbm.at[wid], sidx)
        pltpu.sync_copy(dst_hbm.at[wid], didx)

        def issue(j, b):
            pltpu.async_copy(table_hbm.at[sidx.at[j]], buf_s.at[b], sems[b])
            pltpu.async_copy(table_hbm.at[didx.at[j]], buf_d.at[b], sems[b])
            pltpu.async_copy(lu_src_hbm.at[sidx.at[j]], buf_lu.at[b], sems[b])

        def drain(j, b):
            pltpu.make_async_copy(table_hbm.at[sidx.at[j]], buf_s.at[b],
                                  sems[b]).wait()
            pltpu.make_async_copy(table_hbm.at[didx.at[j]], buf_d.at[b],
                                  sems[b]).wait()
            pltpu.make_async_copy(lu_src_hbm.at[sidx.at[j]], buf_lu.at[b],
                                  sems[b]).wait()

        issue(0, 0)

        def handle(j, b):
            @pl.when(j + 1 < NCHUNK)
            def _():
                issue(j + 1, 1 - b)

            drain(j, b)
            off = base + j * C
            pltpu.sync_copy(buf_s.at[b], srcm_out.at[pl.ds(off, C), :])
            pltpu.sync_copy(buf_d.at[b], dstm_out.at[pl.ds(off, C), :])
            pltpu.sync_copy(buf_lu.at[b], lu_out.at[pl.ds(off, C)])

        def body(i, carry):
            handle(2 * i, 0)
            handle(2 * i + 1, 1)
            return carry

        lax.fori_loop(0, NCHUNK // 2, body, 0)

    return k(src2, dst2, mem_hbm, lu_hbm)


_INV2PI = 0.15915494309189535
_CW1 = 6.283203125
_CW2 = -1.7821788787841797e-05
_CW3 = 3.968374e-09
_COS_COEF = (1.0, -0.5, 0.041666664, -0.0013888867, 2.480069e-05,
             -2.7536993e-07, 2.0620732e-09, -9.774959e-12)


def _fast_cos(x):
    k = lax.round(x * _INV2PI, lax.RoundingMethod.TO_NEAREST_EVEN)
    r = x - k * _CW1
    r = r - k * _CW2
    r = r - k * _CW3
    u = r * r
    acc = jnp.full_like(u, _COS_COEF[7])
    for c in _COS_COEF[6::-1]:
        acc = acc * u + c
    return acc


def _tc_body(srcg, dstg, lug, tt, tw, tb, w1s, w1d, w1t, b1r, w2, b2r, out):
    delta = tt[0] - lug[0]
    arg = lax.dot_general(delta, tw[...], (((0,), (0,)), ((), ())),
                          preferred_element_type=jnp.float32,
                          precision=lax.Precision.HIGHEST)
    enc = _fast_cos(arg + tb[...])
    h = (jnp.dot(srcg[...], w1s[...], preferred_element_type=jnp.float32)
         + jnp.dot(dstg[...], w1d[...], preferred_element_type=jnp.float32)
         + jnp.dot(enc, w1t[...], preferred_element_type=jnp.float32)
         + b1r[...])
    h = jnp.maximum(h, 0.0)
    out[...] = lax.dot_general(w2[...], h, (((0,), (1,)), ((), ())),
                               preferred_element_type=jnp.float32,
                               precision=lax.Precision.HIGHEST) + b2r[...]


def kernel(src, dst, t, edge_attr, memory, last_update, time_W, time_b,
           W1, b1, W2, b2):
    del edge_attr

    pad = B_PAD - B
    nblk = B_PAD // TB
    src_p = jnp.pad(src, (0, pad)).reshape(NW, NCHUNK, C)
    dst_p = jnp.pad(dst, (0, pad)).reshape(NW, NCHUNK, C)
    t_p = jnp.pad(t, (0, pad)).reshape(nblk, 1, TB)

    src_mem, dst_mem, lu_src = _sc_gather(src_p, dst_p, memory, last_update)
    lu_src = lu_src.reshape(nblk, 1, TB)

    out = pl.pallas_call(
        _tc_body,
        grid=(nblk,),
        in_specs=[
            pl.BlockSpec((TB, MEM_DIM), lambda i: (i, 0)),
            pl.BlockSpec((TB, MEM_DIM), lambda i: (i, 0)),
            pl.BlockSpec((1, 1, TB), lambda i: (i, 0, 0)),
            pl.BlockSpec((1, 1, TB), lambda i: (i, 0, 0)),
            pl.BlockSpec((1, TIME_DIM), lambda i: (0, 0)),
            pl.BlockSpec((1, TIME_DIM), lambda i: (0, 0)),
            pl.BlockSpec((MEM_DIM, HIDDEN), lambda i: (0, 0)),
            pl.BlockSpec((MEM_DIM, HIDDEN), lambda i: (0, 0)),
            pl.BlockSpec((TIME_DIM, HIDDEN), lambda i: (0, 0)),
            pl.BlockSpec((1, HIDDEN), lambda i: (0, 0)),
            pl.BlockSpec((HIDDEN, OUT), lambda i: (0, 0)),
            pl.BlockSpec((OUT, 1), lambda i: (0, 0)),
        ],
        out_specs=pl.BlockSpec((OUT, TB), lambda i: (0, i)),
        out_shape=jax.ShapeDtypeStruct((OUT, B_PAD), jnp.float32),
    )(
        src_mem, dst_mem, lu_src, t_p,
        time_W, time_b.reshape(1, TIME_DIM),
        W1[:MEM_DIM], W1[MEM_DIM:2 * MEM_DIM], W1[2 * MEM_DIM:],
        b1.reshape(1, HIDDEN), W2, b2.reshape(OUT, 1),
    )
    return out[:, :B].T


# speedup vs baseline: 1.5518x; 1.5518x over previous
import functools

import jax
import jax.numpy as jnp
from jax import lax
from jax.experimental import pallas as pl
from jax.experimental.pallas import tpu as pltpu
from jax.experimental.pallas import tpu_sc as plsc

NUM_NODES = 100000
MEM_DIM = 128
TIME_DIM = 100
B = 100000
HIDDEN = 100
OUT = 3

NC = 2
NS = 16
NW = NC * NS
C = 112
NCHUNK = 28
CH = C * NCHUNK
B_PAD = CH * NW

TB = 2048


def _sc_gather(src2, dst2, mem_hbm, lu_hbm):
    mesh = plsc.VectorSubcoreMesh(core_axis_name="c", subcore_axis_name="s")

    @functools.partial(
        pl.kernel,
        mesh=mesh,
        out_type=[
            jax.ShapeDtypeStruct((B_PAD, MEM_DIM), jnp.float32),
            jax.ShapeDtypeStruct((B_PAD, MEM_DIM), jnp.float32),
            jax.ShapeDtypeStruct((B_PAD,), jnp.float32),
        ],
        scratch_types=[
            pltpu.VMEM((NCHUNK, C), jnp.int32),
            pltpu.VMEM((NCHUNK, C), jnp.int32),
            pltpu.VMEM((2, C, MEM_DIM), jnp.float32),
            pltpu.VMEM((2, C, MEM_DIM), jnp.float32),
            pltpu.VMEM((2, C), jnp.float32),
            pltpu.SemaphoreType.DMA,
            pltpu.SemaphoreType.DMA,
        ],
    )
    def k(src_hbm, dst_hbm, table_hbm, lu_src_hbm, srcm_out, dstm_out, lu_out,
          sidx, didx, buf_s, buf_d, buf_lu, sem0, sem1):
        cid = lax.axis_index("c")
        sid = lax.axis_index("s")
        wid = sid * NC + cid
        base = wid * CH
        sems = (sem0, sem1)

        pltpu.sync_copy(src_hbm.at[wid], sidx)
        pltpu.sync_copy(dst_hbm.at[wid], didx)

        def issue(j, b):
            pltpu.async_copy(table_hbm.at[sidx.at[j]], buf_s.at[b], sems[b])
            pltpu.async_copy(table_hbm.at[didx.at[j]], buf_d.at[b], sems[b])
            pltpu.async_copy(lu_src_hbm.at[sidx.at[j]], buf_lu.at[b], sems[b])

        def drain(j, b):
            pltpu.make_async_copy(table_hbm.at[sidx.at[j]], buf_s.at[b],
                                  sems[b]).wait()
            pltpu.make_async_copy(table_hbm.at[didx.at[j]], buf_d.at[b],
                                  sems[b]).wait()
            pltpu.make_async_copy(lu_src_hbm.at[sidx.at[j]], buf_lu.at[b],
                                  sems[b]).wait()

        issue(0, 0)

        def handle(j, b):
            @pl.when(j + 1 < NCHUNK)
            def _():
                issue(j + 1, 1 - b)

            drain(j, b)
            off = base + j * C
            pltpu.sync_copy(buf_s.at[b], srcm_out.at[pl.ds(off, C), :])
            pltpu.sync_copy(buf_d.at[b], dstm_out.at[pl.ds(off, C), :])
            pltpu.sync_copy(buf_lu.at[b], lu_out.at[pl.ds(off, C)])

        def body(i, carry):
            handle(2 * i, 0)
            handle(2 * i + 1, 1)
            return carry

        lax.fori_loop(0, NCHUNK // 2, body, 0)

    return k(src2, dst2, mem_hbm, lu_hbm)


_INV2PI = 0.15915494309189535
_CW1 = 6.283203125
_CW2 = -1.7821788787841797e-05
_CW3 = 3.968374e-09
_COS_COEF = (1.0, -0.5, 0.041666664, -0.0013888867, 2.480069e-05,
             -2.7536993e-07, 2.0620732e-09, -9.774959e-12)


def _fast_cos(x):
    k = lax.round(x * _INV2PI, lax.RoundingMethod.TO_NEAREST_EVEN)
    r = x - k * _CW1
    r = r - k * _CW2
    r = r - k * _CW3
    u = r * r
    acc = jnp.full_like(u, _COS_COEF[7])
    for c in _COS_COEF[6::-1]:
        acc = acc * u + c
    return acc


def _tc_body(srcg, dstg, lug, tt, tw, tb, w1s, w1d, w1t, b1r, w2, b2r, out):
    delta = tt[0] - lug[0]
    encT = _fast_cos(tw[...] * delta + tb[...])
    h = (jnp.dot(srcg[...], w1s[...], preferred_element_type=jnp.float32)
         + jnp.dot(dstg[...], w1d[...], preferred_element_type=jnp.float32)
         + lax.dot_general(encT, w1t[...], (((0,), (0,)), ((), ())),
                           preferred_element_type=jnp.float32)
         + b1r[...])
    h = jnp.maximum(h, 0.0)
    out[...] = lax.dot_general(w2[...], h, (((0,), (1,)), ((), ())),
                               preferred_element_type=jnp.float32) + b2r[...]


def kernel(src, dst, t, edge_attr, memory, last_update, time_W, time_b,
           W1, b1, W2, b2):
    del edge_attr

    pad = B_PAD - B
    nblk = B_PAD // TB
    src_p = jnp.pad(src, (0, pad)).reshape(NW, NCHUNK, C)
    dst_p = jnp.pad(dst, (0, pad)).reshape(NW, NCHUNK, C)
    t_p = jnp.pad(t, (0, pad)).reshape(nblk, 1, TB)

    src_mem, dst_mem, lu_src = _sc_gather(src_p, dst_p, memory, last_update)
    lu_src = lu_src.reshape(nblk, 1, TB)

    out = pl.pallas_call(
        _tc_body,
        grid=(nblk,),
        in_specs=[
            pl.BlockSpec((TB, MEM_DIM), lambda i: (i, 0)),
            pl.BlockSpec((TB, MEM_DIM), lambda i: (i, 0)),
            pl.BlockSpec((1, 1, TB), lambda i: (i, 0, 0)),
            pl.BlockSpec((1, 1, TB), lambda i: (i, 0, 0)),
            pl.BlockSpec((TIME_DIM, 1), lambda i: (0, 0)),
            pl.BlockSpec((TIME_DIM, 1), lambda i: (0, 0)),
            pl.BlockSpec((MEM_DIM, HIDDEN), lambda i: (0, 0)),
            pl.BlockSpec((MEM_DIM, HIDDEN), lambda i: (0, 0)),
            pl.BlockSpec((TIME_DIM, HIDDEN), lambda i: (0, 0)),
            pl.BlockSpec((1, HIDDEN), lambda i: (0, 0)),
            pl.BlockSpec((HIDDEN, OUT), lambda i: (0, 0)),
            pl.BlockSpec((OUT, 1), lambda i: (0, 0)),
        ],
        out_specs=pl.BlockSpec((OUT, TB), lambda i: (0, i)),
        out_shape=jax.ShapeDtypeStruct((OUT, B_PAD), jnp.float32),
    )(
        src_mem, dst_mem, lu_src, t_p,
        time_W.reshape(TIME_DIM, 1), time_b.reshape(TIME_DIM, 1),
        W1[:MEM_DIM], W1[MEM_DIM:2 * MEM_DIM], W1[2 * MEM_DIM:],
        b1.reshape(1, HIDDEN), W2, b2.reshape(OUT, 1),
    )
    return out[:, :B].T
